# baseline (speedup 1.0000x reference)
import functools

import jax
import jax.numpy as jnp
from jax import lax
from jax.experimental import pallas as pl
from jax.experimental.pallas import tpu as pltpu
from jax.experimental.pallas import tpu_sc as plsc

_B = 4
_S = 2048
_E = 128
_EPS = 1e-12
_L = 16
_NE = _E // _L

_NC = 2
_NS = 16
_NW = _NC * _NS
_NTOK = _B * _S
_TPW = _NTOK // _NW
_ICH = 128
_NCH = _TPW // _ICH

_INV_E = 1.0 / _E
_MAGIC = 0x5F3759DF


def _tree_sum(vs):
    while len(vs) > 1:
        vs = [a + b for a, b in zip(vs[::2], vs[1::2])]
    return vs[0]


def _fused_sc(ids3, tt, wemb, pemb, temb, gb):
    mesh = plsc.VectorSubcoreMesh(core_axis_name="c", subcore_axis_name="s")

    @functools.partial(
        pl.kernel,
        out_type=jax.ShapeDtypeStruct((_NTOK, _E), jnp.float32),
        mesh=mesh,
        compiler_params=pltpu.CompilerParams(needs_layout_passes=False),
        scratch_types=[
            pltpu.VMEM((_NCH, _ICH), jnp.int32),
            pltpu.VMEM((_TPW,), jnp.int32),
            pltpu.VMEM((_TPW, _E), jnp.float32),
            pltpu.VMEM((_TPW, _E), jnp.float32),
            pltpu.VMEM((2, _E), jnp.float32),
            pltpu.VMEM((2, _E), jnp.float32),
            pltpu.SemaphoreType.DMA,
        ],
    )
    def k(ids_hbm, tt_hbm, wemb_hbm, pemb_hbm, temb_hbm, gb_hbm, out_hbm,
          idx_v, tt_v, rows_v, pos_v, te_v, gb_v, sem):
        wid = lax.axis_index("s") * _NC + lax.axis_index("c")
        base = wid * _TPW
        pbase = lax.rem(base, _S)

        aux = [
            pltpu.async_copy(pemb_hbm.at[pl.ds(pbase, _TPW)], pos_v, sem),
            pltpu.async_copy(tt_hbm.at[pl.ds(base, _TPW)], tt_v, sem),
            pltpu.async_copy(temb_hbm, te_v, sem),
            pltpu.async_copy(gb_hbm, gb_v, sem),
        ]
        pltpu.sync_copy(ids_hbm.at[wid], idx_v)
        gathers = [
            pltpu.async_copy(wemb_hbm.at[idx_v.at[j]],
                             rows_v.at[pl.ds(j * _ICH, _ICH)], sem)
            for j in range(_NCH)
        ]
        for cp in aux + gathers:
            cp.wait()

        r0 = [te_v[0, pl.ds(e * _L, _L)] for e in range(_NE)]
        dd = [te_v[1, pl.ds(e * _L, _L)] - r0[e] for e in range(_NE)]
        gg = [gb_v[0, pl.ds(e * _L, _L)] for e in range(_NE)]
        bb = [gb_v[1, pl.ds(e * _L, _L)] for e in range(_NE)]

        @plsc.parallel_loop(0, _TPW // _L, unroll=1)
        def _(g):
            tlf = tt_v[pl.ds(g * _L, _L)].astype(jnp.float32)
            for j in range(_L):
                i = g * _L + j
                tf = jnp.full((_L,), tlf[j])
                s = []
                for e in range(_NE):
                    w = rows_v[i, pl.ds(e * _L, _L)]
                    p = pos_v[i, pl.ds(e * _L, _L)]
                    s.append(w + p + r0[e] + tf * dd[e])
                tot = _tree_sum(s)
                sq = _tree_sum([x * x for x in s])
                mean = jnp.sum(tot) * _INV_E
                var = jnp.sum(sq) * _INV_E - mean * mean + _EPS
                vv = jnp.full((_L,), var)
                iv = plsc.bitcast(vv, jnp.int32)
                y = plsc.bitcast(jnp.int32(_MAGIC) - (iv >> 1), jnp.float32)
                h = 0.5 * vv
                for _ in range(3):
                    y = y * (1.5 - h * y * y)
                mv = jnp.full((_L,), mean)
                for e in range(_NE):
                    rows_v[i, pl.ds(e * _L, _L)] = (s[e] - mv) * y * gg[e] + bb[e]

        pltpu.sync_copy(rows_v, out_hbm.at[pl.ds(base, _TPW)])

    return k(ids3, tt, wemb, pemb, temb, gb)


def kernel(input_ids, token_type_ids, word_embeddings, position_embeddings,
           token_type_embeddings, gamma, beta):
    ids3 = input_ids.astype(jnp.int32).reshape(_NW, _NCH, _ICH)
    tt = token_type_ids.astype(jnp.int32).reshape(_NTOK)
    gb = jnp.stack([gamma, beta])
    rows = _fused_sc(ids3, tt, word_embeddings, position_embeddings,
                     token_type_embeddings, gb)
    return rows.reshape(_B, _S, _E)
